# baseline (speedup 1.0000x reference)
import functools

import jax
import jax.numpy as jnp
from jax import lax
from jax.experimental import pallas as pl
from jax.experimental.pallas import tpu as pltpu
from jax.experimental.pallas import tpu_sc as plsc

_N = 10000
_F = 128
_K = 16
_NP = 10240
_BLK = 128
_NW = 32
_CH = 128


def _simtopk_body(emb_blk_ref, emb_full_ref, vals_ref, idx_ref):
    s = jax.lax.dot_general(
        emb_blk_ref[...], emb_full_ref[...],
        (((1,), (1,)), ((), ())),
        preferred_element_type=jnp.float32,
    )
    col = jax.lax.broadcasted_iota(jnp.int32, s.shape, 1)
    s = jnp.where(col < _N, s, -jnp.inf)
    for k in range(_K):
        m = jnp.max(s, axis=1, keepdims=True)
        cand = jnp.where(s == m, col, _NP)
        am = jnp.min(cand, axis=1, keepdims=True)
        vals_ref[:, k] = m[:, 0]
        idx_ref[:, k] = am[:, 0]
        s = jnp.where(col == am, -jnp.inf, s)


def _sim_topk(emb):
    emb_p = jnp.zeros((_NP, _F), dtype=jnp.float32).at[:_N].set(emb)
    vals, idx = pl.pallas_call(
        _simtopk_body,
        grid=(_NP // _BLK,),
        in_specs=[
            pl.BlockSpec((_BLK, _F), lambda i: (i, 0)),
            pl.BlockSpec((_NP, _F), lambda i: (0, 0)),
        ],
        out_specs=[
            pl.BlockSpec((_BLK, _K), lambda i: (i, 0)),
            pl.BlockSpec((_BLK, _K), lambda i: (i, 0)),
        ],
        out_shape=[
            jax.ShapeDtypeStruct((_NP, _K), jnp.float32),
            jax.ShapeDtypeStruct((_NP, _K), jnp.int32),
        ],
    )(emb_p, emb_p)
    return vals[:_N], idx[:_N]


def _sc_gather(table, idx):
    B = idx.shape[0]
    D0 = table.shape[1]
    if D0 % 128 != 0:
        table = jnp.pad(table, ((0, 0), (0, 128 - D0 % 128)))
    D = table.shape[1]
    per_w = B // _NW
    assert per_w * _NW == B and per_w % 8 == 0
    n_full = per_w // _CH
    tail = per_w - n_full * _CH
    mesh = plsc.VectorSubcoreMesh(core_axis_name="c", subcore_axis_name="s")

    @functools.partial(
        pl.kernel, mesh=mesh,
        out_type=jax.ShapeDtypeStruct((B, D), jnp.float32),
        scratch_types=[
            pltpu.VMEM((_CH,), jnp.int32),
            pltpu.VMEM((_CH, D), jnp.float32),
            pltpu.SemaphoreType.DMA,
        ],
    )
    def k(table_hbm, idx_hbm, out_hbm, idx_v, buf, sem):
        wid = lax.axis_index("s") * 2 + lax.axis_index("c")
        base = wid * per_w

        def chunk(off):
            pltpu.sync_copy(idx_hbm.at[pl.ds(off, _CH)], idx_v)
            pltpu.async_copy(table_hbm.at[idx_v], buf, sem).wait()
            pltpu.sync_copy(buf, out_hbm.at[pl.ds(off, _CH)])

        def body(j, c):
            chunk(base + j * _CH)
            return c

        lax.fori_loop(0, n_full, body, 0)
        if tail:
            chunk(base + per_w - _CH)

    out = k(table, idx)
    return out[:, :D0] if D0 != D else out


def _spmm_sc(indices, values, x):
    gathered = _sc_gather(x, indices[1]) * values[:, None]
    return jax.ops.segment_sum(gathered, indices[0], num_segments=_N)


def kernel(input, adj_indices, adj_values, W_diag1, W_diag2, W1, b1, W2, b2):
    deg0 = jax.ops.segment_sum(adj_values, adj_indices[0], num_segments=_N)
    inv0 = 1.0 / (jnp.sqrt(deg0) + 1e-10)
    norm_vals = (adj_values * jnp.take(inv0, adj_indices[0], mode="clip")
                 * jnp.take(inv0, adj_indices[1], mode="clip"))
    h = jnp.tanh(_spmm_sc(adj_indices, norm_vals, input * W_diag1))
    emb = _spmm_sc(adj_indices, norm_vals, h * W_diag2)
    nrm = jnp.linalg.norm(emb, axis=1, keepdims=True)
    emb = emb / jnp.maximum(nrm, 1e-12)
    vals, idx = _sim_topk(emb)
    rows = jnp.repeat(jnp.arange(_N, dtype=jnp.int32), _K)
    idx_flat = idx.reshape(-1)
    inds = jnp.stack([rows, idx_flat])
    inds_sym = jnp.concatenate([inds, jnp.stack([inds[1], inds[0]])], axis=1)
    vals_flat = vals.reshape(-1)
    vals_sym = jnp.concatenate([vals_flat, vals_flat])
    new_inds = jnp.concatenate([adj_indices.astype(jnp.int32), inds_sym], axis=1)
    new_vals = jnp.concatenate([adj_values, vals_sym])
    deg_new = (deg0 + jnp.sum(vals, axis=1)
               + jax.ops.segment_sum(vals_flat, idx_flat, num_segments=_N))
    inv = 1.0 / (jnp.sqrt(deg_new) + 1e-10)

    def spmm_new(z):
        zi = z * inv[:, None]
        part_o = jax.ops.segment_sum(
            _sc_gather(zi, adj_indices[1]) * adj_values[:, None],
            adj_indices[0], num_segments=_N)
        zg = _sc_gather(zi, idx_flat).reshape(_N, _K, -1)
        part_g = jnp.sum(vals[:, :, None] * zg, axis=1)
        part_t = jax.ops.segment_sum(
            jnp.broadcast_to(zi[:, None, :], (_N, _K, zi.shape[1]))
            .reshape(_N * _K, -1) * vals_flat[:, None],
            idx_flat, num_segments=_N)
        return inv[:, None] * (part_o + part_g + part_t)

    h1 = jax.nn.relu(spmm_new(input @ W1 + b1))
    x_out = spmm_new(h1 @ W2 + b2)
    return (x_out, inds_sym, vals_sym, new_inds, new_vals)


# speedup vs baseline: 1.0151x; 1.0151x over previous
import functools

import jax
import jax.numpy as jnp
from jax import lax
from jax.experimental import pallas as pl
from jax.experimental.pallas import tpu as pltpu
from jax.experimental.pallas import tpu_sc as plsc

_N = 10000
_F = 128
_K = 16
_NP = 10240
_BLK = 128
_NW = 32
_CH = 128


def _simtopk_body(emb_blk_ref, emb_full_ref, vals_ref, idx_ref):
    s = jax.lax.dot_general(
        emb_blk_ref[...], emb_full_ref[...],
        (((1,), (1,)), ((), ())),
        preferred_element_type=jnp.float32,
    )
    col = jax.lax.broadcasted_iota(jnp.int32, s.shape, 1)
    s = jnp.where(col < _N, s, -jnp.inf)
    for k in range(_K):
        m = jnp.max(s, axis=1, keepdims=True)
        cand = jnp.where(s == m, col, _NP)
        am = jnp.min(cand, axis=1, keepdims=True)
        vals_ref[:, k] = m[:, 0]
        idx_ref[:, k] = am[:, 0]
        s = jnp.where(col == am, -jnp.inf, s)


def _sim_topk(emb):
    emb_p = jnp.zeros((_NP, _F), dtype=jnp.float32).at[:_N].set(emb)
    vals, idx = pl.pallas_call(
        _simtopk_body,
        grid=(_NP // _BLK,),
        in_specs=[
            pl.BlockSpec((_BLK, _F), lambda i: (i, 0)),
            pl.BlockSpec((_NP, _F), lambda i: (0, 0)),
        ],
        out_specs=[
            pl.BlockSpec((_BLK, _K), lambda i: (i, 0)),
            pl.BlockSpec((_BLK, _K), lambda i: (i, 0)),
        ],
        out_shape=[
            jax.ShapeDtypeStruct((_NP, _K), jnp.float32),
            jax.ShapeDtypeStruct((_NP, _K), jnp.int32),
        ],
    )(emb_p, emb_p)
    return vals[:_N], idx[:_N]


def _sc_gather(table, idx):
    B = idx.shape[0]
    D0 = table.shape[1]
    if D0 % 128 != 0:
        table = jnp.pad(table, ((0, 0), (0, 128 - D0 % 128)))
    D = table.shape[1]
    per_w = B // _NW
    assert per_w * _NW == B and per_w % 8 == 0
    n_full = per_w // _CH
    tail = per_w - n_full * _CH
    mesh = plsc.VectorSubcoreMesh(core_axis_name="c", subcore_axis_name="s")

    n_pairs = n_full // 2
    odd = n_full - n_pairs * 2

    @functools.partial(
        pl.kernel, mesh=mesh,
        out_type=jax.ShapeDtypeStruct((B, D), jnp.float32),
        scratch_types=[
            pltpu.VMEM((_CH,), jnp.int32),
            pltpu.VMEM((_CH,), jnp.int32),
            pltpu.VMEM((_CH, D), jnp.float32),
            pltpu.VMEM((_CH, D), jnp.float32),
            pltpu.SemaphoreType.DMA,
            pltpu.SemaphoreType.DMA,
        ],
    )
    def k(table_hbm, idx_hbm, out_hbm, idx0, idx1, buf0, buf1, sem0, sem1):
        wid = lax.axis_index("s") * 2 + lax.axis_index("c")
        base = wid * per_w
        idxs, bufs, sems = (idx0, idx1), (buf0, buf1), (sem0, sem1)

        def chunk(off):
            pltpu.sync_copy(idx_hbm.at[pl.ds(off, _CH)], idx0)
            pltpu.async_copy(table_hbm.at[idx0], buf0, sem0).wait()
            pltpu.sync_copy(buf0, out_hbm.at[pl.ds(off, _CH)])

        def pair(j2, c):
            offs = [base + (j2 * 2 + b) * _CH for b in range(2)]
            cps = []
            for b in range(2):
                pltpu.sync_copy(idx_hbm.at[pl.ds(offs[b], _CH)], idxs[b])
                cps.append(pltpu.async_copy(table_hbm.at[idxs[b]], bufs[b], sems[b]))
            for b in range(2):
                cps[b].wait()
                pltpu.sync_copy(bufs[b], out_hbm.at[pl.ds(offs[b], _CH)])
            return c

        lax.fori_loop(0, n_pairs, pair, 0)
        if odd:
            chunk(base + (n_full - 1) * _CH)
        if tail:
            chunk(base + per_w - _CH)

    out = k(table, idx)
    return out[:, :D0] if D0 != D else out


def _spmm_sc(indices, values, x):
    gathered = _sc_gather(x, indices[1]) * values[:, None]
    return jax.ops.segment_sum(gathered, indices[0], num_segments=_N)


def kernel(input, adj_indices, adj_values, W_diag1, W_diag2, W1, b1, W2, b2):
    deg0 = jax.ops.segment_sum(adj_values, adj_indices[0], num_segments=_N)
    inv0 = 1.0 / (jnp.sqrt(deg0) + 1e-10)
    norm_vals = (adj_values * jnp.take(inv0, adj_indices[0], mode="clip")
                 * jnp.take(inv0, adj_indices[1], mode="clip"))
    h = jnp.tanh(_spmm_sc(adj_indices, norm_vals, input * W_diag1))
    emb = _spmm_sc(adj_indices, norm_vals, h * W_diag2)
    nrm = jnp.linalg.norm(emb, axis=1, keepdims=True)
    emb = emb / jnp.maximum(nrm, 1e-12)
    vals, idx = _sim_topk(emb)
    rows = jnp.repeat(jnp.arange(_N, dtype=jnp.int32), _K)
    idx_flat = idx.reshape(-1)
    inds = jnp.stack([rows, idx_flat])
    inds_sym = jnp.concatenate([inds, jnp.stack([inds[1], inds[0]])], axis=1)
    vals_flat = vals.reshape(-1)
    vals_sym = jnp.concatenate([vals_flat, vals_flat])
    new_inds = jnp.concatenate([adj_indices.astype(jnp.int32), inds_sym], axis=1)
    new_vals = jnp.concatenate([adj_values, vals_sym])
    deg_new = (deg0 + jnp.sum(vals, axis=1)
               + jax.ops.segment_sum(vals_flat, idx_flat, num_segments=_N))
    inv = 1.0 / (jnp.sqrt(deg_new) + 1e-10)
    gidx = jnp.concatenate([adj_indices[1], idx_flat])

    def spmm_new(z):
        zi = z * inv[:, None]
        g = _sc_gather(zi, gidx)
        part_o = jax.ops.segment_sum(
            g[:adj_values.shape[0]] * adj_values[:, None],
            adj_indices[0], num_segments=_N)
        zg = g[adj_values.shape[0]:].reshape(_N, _K, -1)
        part_g = jnp.sum(vals[:, :, None] * zg, axis=1)
        part_t = jax.ops.segment_sum(
            jnp.broadcast_to(zi[:, None, :], (_N, _K, zi.shape[1]))
            .reshape(_N * _K, -1) * vals_flat[:, None],
            idx_flat, num_segments=_N)
        return inv[:, None] * (part_o + part_g + part_t)

    h1 = jax.nn.relu(spmm_new(input @ W1 + b1))
    x_out = spmm_new(h1 @ W2 + b2)
    return (x_out, inds_sym, vals_sym, new_inds, new_vals)


# speedup vs baseline: 1.0270x; 1.0117x over previous
import functools

import jax
import jax.numpy as jnp
from jax import lax
from jax.experimental import pallas as pl
from jax.experimental.pallas import tpu as pltpu
from jax.experimental.pallas import tpu_sc as plsc

_N = 10000
_F = 128
_K = 16
_BLK = 200
_NW = 32
_CH = 128


def _simtopk_body(emb_blk_ref, emb_full_ref, vals_ref, idx_ref):
    s = jax.lax.dot_general(
        emb_blk_ref[...], emb_full_ref[...],
        (((1,), (1,)), ((), ())),
        preferred_element_type=jnp.float32,
    )
    col = jax.lax.broadcasted_iota(jnp.int32, s.shape, 1)
    for k in range(_K):
        m = jnp.max(s, axis=1, keepdims=True)
        cand = jnp.where(s == m, col, _N)
        am = jnp.min(cand, axis=1, keepdims=True)
        vals_ref[:, k] = m[:, 0]
        idx_ref[:, k] = am[:, 0]
        s = jnp.where(col == am, -jnp.inf, s)


def _sim_topk(emb):
    vals, idx = pl.pallas_call(
        _simtopk_body,
        grid=(_N // _BLK,),
        in_specs=[
            pl.BlockSpec((_BLK, _F), lambda i: (i, 0)),
            pl.BlockSpec((_N, _F), lambda i: (0, 0)),
        ],
        out_specs=[
            pl.BlockSpec((_BLK, _K), lambda i: (i, 0)),
            pl.BlockSpec((_BLK, _K), lambda i: (i, 0)),
        ],
        out_shape=[
            jax.ShapeDtypeStruct((_N, _K), jnp.float32),
            jax.ShapeDtypeStruct((_N, _K), jnp.int32),
        ],
    )(emb, emb)
    return vals, idx


def _sc_gather(table, idx):
    B = idx.shape[0]
    D0 = table.shape[1]
    if D0 % 128 != 0:
        table = jnp.pad(table, ((0, 0), (0, 128 - D0 % 128)))
    D = table.shape[1]
    per_w = B // _NW
    assert per_w * _NW == B and per_w % 8 == 0
    n_full = per_w // _CH
    tail = per_w - n_full * _CH
    mesh = plsc.VectorSubcoreMesh(core_axis_name="c", subcore_axis_name="s")

    n_pairs = n_full // 2
    odd = n_full - n_pairs * 2

    @functools.partial(
        pl.kernel, mesh=mesh,
        out_type=jax.ShapeDtypeStruct((B, D), jnp.float32),
        scratch_types=[
            pltpu.VMEM((_CH,), jnp.int32),
            pltpu.VMEM((_CH,), jnp.int32),
            pltpu.VMEM((_CH, D), jnp.float32),
            pltpu.VMEM((_CH, D), jnp.float32),
            pltpu.SemaphoreType.DMA,
            pltpu.SemaphoreType.DMA,
        ],
    )
    def k(table_hbm, idx_hbm, out_hbm, idx0, idx1, buf0, buf1, sem0, sem1):
        wid = lax.axis_index("s") * 2 + lax.axis_index("c")
        base = wid * per_w
        idxs, bufs, sems = (idx0, idx1), (buf0, buf1), (sem0, sem1)

        def chunk(off):
            pltpu.sync_copy(idx_hbm.at[pl.ds(off, _CH)], idx0)
            pltpu.async_copy(table_hbm.at[idx0], buf0, sem0).wait()
            pltpu.sync_copy(buf0, out_hbm.at[pl.ds(off, _CH)])

        def pair(j2, c):
            offs = [base + (j2 * 2 + b) * _CH for b in range(2)]
            cps = []
            for b in range(2):
                pltpu.sync_copy(idx_hbm.at[pl.ds(offs[b], _CH)], idxs[b])
                cps.append(pltpu.async_copy(table_hbm.at[idxs[b]], bufs[b], sems[b]))
            for b in range(2):
                cps[b].wait()
                pltpu.sync_copy(bufs[b], out_hbm.at[pl.ds(offs[b], _CH)])
            return c

        lax.fori_loop(0, n_pairs, pair, 0)
        if odd:
            chunk(base + (n_full - 1) * _CH)
        if tail:
            chunk(base + per_w - _CH)

    out = k(table, idx)
    return out[:, :D0] if D0 != D else out


def _spmm_sc(indices, values, x):
    gathered = _sc_gather(x, indices[1]) * values[:, None]
    return jax.ops.segment_sum(gathered, indices[0], num_segments=_N)


def kernel(input, adj_indices, adj_values, W_diag1, W_diag2, W1, b1, W2, b2):
    deg0 = jax.ops.segment_sum(adj_values, adj_indices[0], num_segments=_N)
    inv0 = 1.0 / (jnp.sqrt(deg0) + 1e-10)
    norm_vals = (adj_values * jnp.take(inv0, adj_indices[0], mode="clip")
                 * jnp.take(inv0, adj_indices[1], mode="clip"))
    h = jnp.tanh(_spmm_sc(adj_indices, norm_vals, input * W_diag1))
    emb = _spmm_sc(adj_indices, norm_vals, h * W_diag2)
    nrm = jnp.linalg.norm(emb, axis=1, keepdims=True)
    emb = emb / jnp.maximum(nrm, 1e-12)
    vals, idx = _sim_topk(emb)
    rows = jnp.repeat(jnp.arange(_N, dtype=jnp.int32), _K)
    idx_flat = idx.reshape(-1)
    inds = jnp.stack([rows, idx_flat])
    inds_sym = jnp.concatenate([inds, jnp.stack([inds[1], inds[0]])], axis=1)
    vals_flat = vals.reshape(-1)
    vals_sym = jnp.concatenate([vals_flat, vals_flat])
    new_inds = jnp.concatenate([adj_indices.astype(jnp.int32), inds_sym], axis=1)
    new_vals = jnp.concatenate([adj_values, vals_sym])
    deg_new = (deg0 + jnp.sum(vals, axis=1)
               + jax.ops.segment_sum(vals_flat, idx_flat, num_segments=_N))
    inv = 1.0 / (jnp.sqrt(deg_new) + 1e-10)
    gidx = jnp.concatenate([adj_indices[1], idx_flat])

    def spmm_new(z):
        zi = z * inv[:, None]
        g = _sc_gather(zi, gidx)
        part_o = jax.ops.segment_sum(
            g[:adj_values.shape[0]] * adj_values[:, None],
            adj_indices[0], num_segments=_N)
        zg = g[adj_values.shape[0]:].reshape(_N, _K, -1)
        part_g = jnp.sum(vals[:, :, None] * zg, axis=1)
        part_t = jax.ops.segment_sum(
            jnp.broadcast_to(zi[:, None, :], (_N, _K, zi.shape[1]))
            .reshape(_N * _K, -1) * vals_flat[:, None],
            idx_flat, num_segments=_N)
        return inv[:, None] * (part_o + part_g + part_t)

    h1 = jax.nn.relu(spmm_new(input @ W1 + b1))
    x_out = spmm_new(h1 @ W2 + b2)
    return (x_out, inds_sym, vals_sym, new_inds, new_vals)
